# Initial kernel scaffold; baseline (speedup 1.0000x reference)
import jax
import jax.numpy as jnp
from jax.experimental import pallas as pl


def kernel(input_ids, tok_emb, pos_emb):
    raise NotImplementedError("write your pallas kernel here")


# speedup vs baseline: 2.8793x; 2.8793x over previous
import jax
import jax.numpy as jnp
from jax import lax
from jax.experimental import pallas as pl
from jax.experimental.pallas import tpu as pltpu
from jax.experimental.pallas import tpu_sc as plsc

D = 64
SEQ = 200
HALF = 100
LANES = 16

_info = plsc.get_sparse_core_info()
NC, NS = _info.num_cores, _info.num_subcores
NW = NC * NS


def _emb_body(ids_hbm, tok_hbm, pos_hbm, out_hbm, pos_v, idx_v, rows_v, sem):
    wid = lax.axis_index("s") * NC + lax.axis_index("c")
    num_seq = ids_hbm.shape[0] // 2
    seq_per_w = num_seq // NW

    pltpu.sync_copy(pos_hbm.at[pl.ds(0, SEQ)], pos_v)

    def seq_body(j, carry):
        seq = wid * seq_per_w + j
        pltpu.sync_copy(ids_hbm.at[pl.ds(2 * seq, 2)], idx_v)
        for h in range(2):
            pltpu.async_copy(tok_hbm.at[idx_v.at[h]], rows_v.at[h], sem).wait()

        def row_body(r, c2):
            for h in range(2):
                for g in range(D // LANES):
                    sl = pl.ds(g * LANES, LANES)
                    rows_v[h, r, sl] = rows_v[h, r, sl] + pos_v[h * HALF + r, sl]
            return c2

        lax.fori_loop(0, HALF, row_body, 0)
        pltpu.sync_copy(rows_v, out_hbm.at[pl.ds(2 * seq, 2)])
        return carry

    lax.fori_loop(0, seq_per_w, seq_body, 0)


def kernel(input_ids, tok_emb, pos_emb):
    B, S = input_ids.shape
    ids2 = input_ids.reshape(B * S // HALF, HALF).astype(jnp.int32)
    mesh = plsc.VectorSubcoreMesh(core_axis_name="c", subcore_axis_name="s")
    k = pl.kernel(
        _emb_body,
        mesh=mesh,
        out_type=jax.ShapeDtypeStruct((B * S // HALF, HALF, D), jnp.float32),
        scratch_types=[
            pltpu.VMEM((SEQ, D), jnp.float32),
            pltpu.VMEM((2, HALF), jnp.int32),
            pltpu.VMEM((2, HALF, D), jnp.float32),
            pltpu.SemaphoreType.DMA,
        ],
        compiler_params=pltpu.CompilerParams(use_tc_tiling_on_sc=False),
    )
    out = k(ids2, tok_emb, pos_emb)
    return out.reshape(B, S, D)


# speedup vs baseline: 3.9158x; 1.3600x over previous
import jax
import jax.numpy as jnp
from jax import lax
from jax.experimental import pallas as pl
from jax.experimental.pallas import tpu as pltpu
from jax.experimental.pallas import tpu_sc as plsc

D = 64
SEQ = 200
HALF = 100
LANES = 16
K = 4

_info = plsc.get_sparse_core_info()
NC, NS = _info.num_cores, _info.num_subcores
NW = NC * NS


def _emb_body(ids_hbm, tok_hbm, pos_hbm, out_hbm, pos_v, idx_v, gin, gout,
              sem_g, sem_o):
    wid = lax.axis_index("s") * NC + lax.axis_index("c")
    num_seq = ids_hbm.shape[0] // 2
    seq_per_w = num_seq // NW
    first = wid * seq_per_w
    n_outer = seq_per_w // K

    pltpu.sync_copy(pos_hbm.at[pl.ds(0, SEQ)], pos_v)

    def gather_start(b, seq):
        pltpu.sync_copy(ids_hbm.at[pl.ds(2 * seq, 2)], idx_v.at[b])
        for h in range(2):
            pltpu.async_copy(tok_hbm.at[idx_v.at[b].at[h]], gin.at[b].at[h],
                             sem_g.at[b])

    def gather_wait(b, seq):
        for h in range(2):
            pltpu.make_async_copy(tok_hbm.at[idx_v.at[b].at[h]],
                                  gin.at[b].at[h], sem_g.at[b]).wait()

    def store_start(b, seq):
        pltpu.async_copy(gout.at[b], out_hbm.at[pl.ds(2 * seq, 2)], sem_o.at[b])

    def store_wait(b, seq):
        pltpu.make_async_copy(gout.at[b], out_hbm.at[pl.ds(2 * seq, 2)],
                              sem_o.at[b]).wait()

    def compute(b):
        def row_body(r2, c):
            for u in range(2):
                r = r2 * 2 + u
                for h in range(2):
                    for g in range(D // LANES):
                        sl = pl.ds(g * LANES, LANES)
                        gout[b, h, r, sl] = gin[b, h, r, sl] + pos_v[h * HALF + r, sl]
            return c
        lax.fori_loop(0, HALF // 2, row_body, 0)

    for b in range(K):
        gather_start(b, first + b)

    def outer(g, carry):
        for b in range(K):
            seq = first + g * K + b
            gather_wait(b, seq)

            @pl.when(g > 0)
            def _():
                store_wait(b, seq - K)

            compute(b)
            store_start(b, seq)

            @pl.when(g < n_outer - 1)
            def _():
                gather_start(b, seq + K)
        return carry

    lax.fori_loop(0, n_outer, outer, 0)

    for b in range(K):
        store_wait(b, first + (n_outer - 1) * K + b)


def kernel(input_ids, tok_emb, pos_emb):
    B, S = input_ids.shape
    ids2 = input_ids.reshape(B * S // HALF, HALF).astype(jnp.int32)
    mesh = plsc.VectorSubcoreMesh(core_axis_name="c", subcore_axis_name="s")
    k = pl.kernel(
        _emb_body,
        mesh=mesh,
        out_type=jax.ShapeDtypeStruct((B * S // HALF, HALF, D), jnp.float32),
        scratch_types=[
            pltpu.VMEM((SEQ, D), jnp.float32),
            pltpu.VMEM((K, 2, HALF), jnp.int32),
            pltpu.VMEM((K, 2, HALF, D), jnp.float32),
            pltpu.VMEM((K, 2, HALF, D), jnp.float32),
            pltpu.SemaphoreType.DMA((K,)),
            pltpu.SemaphoreType.DMA((K,)),
        ],
        compiler_params=pltpu.CompilerParams(use_tc_tiling_on_sc=False),
    )
    out = k(ids2, tok_emb, pos_emb)
    return out.reshape(B, S, D)


# speedup vs baseline: 3.9208x; 1.0013x over previous
import jax
import jax.numpy as jnp
from jax import lax
from jax.experimental import pallas as pl
from jax.experimental.pallas import tpu as pltpu
from jax.experimental.pallas import tpu_sc as plsc

D = 64
SEQ = 200
HALF = 100
LANES = 16
K = 4

_info = plsc.get_sparse_core_info()
NC, NS = _info.num_cores, _info.num_subcores
NW = NC * NS


def _emb_body(ids_hbm, tok_hbm, pos_hbm, out_hbm, pos_v, idx_v, gin, gout,
              sem_g, sem_o):
    wid = lax.axis_index("s") * NC + lax.axis_index("c")
    num_seq = ids_hbm.shape[0] // 2
    seq_per_w = num_seq // NW
    first = wid * seq_per_w
    n_outer = seq_per_w // K

    pltpu.sync_copy(pos_hbm.at[pl.ds(0, SEQ)], pos_v)

    def gather_start(b, seq):
        pltpu.sync_copy(ids_hbm.at[pl.ds(2 * seq, 2)], idx_v.at[b])
        for h in range(2):
            pltpu.async_copy(tok_hbm.at[idx_v.at[b].at[h]],
                             gin.at[b].at[h], sem_g.at[b])

    def gather_wait(b, seq):
        for h in range(2):
            pltpu.make_async_copy(
                tok_hbm.at[idx_v.at[b].at[h]],
                gin.at[b].at[h], sem_g.at[b]).wait()

    def store_start(b, seq):
        pltpu.async_copy(gout.at[b], out_hbm.at[pl.ds(seq, 1)], sem_o.at[b])

    def store_wait(b, seq):
        pltpu.make_async_copy(gout.at[b], out_hbm.at[pl.ds(seq, 1)],
                              sem_o.at[b]).wait()

    def compute(b):
        def row_body(r2, c):
            for u in range(2):
                r = r2 * 2 + u
                for h in range(2):
                    for g in range(D // LANES):
                        sl = pl.ds(g * LANES, LANES)
                        gout[b, 0, h * HALF + r, sl] = (
                            gin[b, h, r, sl] + pos_v[h * HALF + r, sl])
            return c
        lax.fori_loop(0, HALF // 2, row_body, 0)

    for b in range(K):
        gather_start(b, first + b)

    def outer(g, carry):
        for b in range(K):
            seq = first + g * K + b
            gather_wait(b, seq)

            @pl.when(g > 0)
            def _():
                store_wait(b, seq - K)

            compute(b)
            store_start(b, seq)

            @pl.when(g < n_outer - 1)
            def _():
                gather_start(b, seq + K)
        return carry

    lax.fori_loop(0, n_outer, outer, 0)

    for b in range(K):
        store_wait(b, first + (n_outer - 1) * K + b)


def kernel(input_ids, tok_emb, pos_emb):
    B, S = input_ids.shape
    ids = input_ids.reshape(B * S // HALF, HALF).astype(jnp.int32)
    mesh = plsc.VectorSubcoreMesh(core_axis_name="c", subcore_axis_name="s")
    k = pl.kernel(
        _emb_body,
        mesh=mesh,
        out_type=jax.ShapeDtypeStruct((B, S, D), jnp.float32),
        scratch_types=[
            pltpu.VMEM((SEQ, D), jnp.float32),
            pltpu.VMEM((K, 2, HALF), jnp.int32),
            pltpu.VMEM((K, 2, HALF, D), jnp.float32),
            pltpu.VMEM((K, 1, SEQ, D), jnp.float32),
            pltpu.SemaphoreType.DMA((K,)),
            pltpu.SemaphoreType.DMA((K,)),
        ],
        compiler_params=pltpu.CompilerParams(use_tc_tiling_on_sc=False),
    )
    return k(ids, tok_emb, pos_emb)
